# Initial kernel scaffold; baseline (speedup 1.0000x reference)
import jax
import jax.numpy as jnp
from jax.experimental import pallas as pl


def kernel(x, edge_index, edge_attr, batch):
    raise NotImplementedError("write your pallas kernel here")


# speedup vs baseline: 3.8662x; 3.8662x over previous
import functools

import jax
import jax.numpy as jnp
from jax import lax
from jax.experimental import pallas as pl
from jax.experimental.pallas import tpu as pltpu
from jax.experimental.pallas import tpu_sc as plsc

N = 10000
E = 320000
D = 128
DH = 64
NP = 10240
NS = 16
CH = 128
EPT = 160
E_PAD = NS * EPT * CH
ROWS_PER_TILE = NP // NS
NBLK = ROWS_PER_TILE // CH


def _spmm_body(h0, h1, srcb, dstb, zblk, out0, out1, src_v, dst_v, rows, acc, sem):
    c = lax.axis_index("c")
    s = lax.axis_index("s")
    rbase = s * ROWS_PER_TILE
    for k in range(NBLK):
        pltpu.sync_copy(zblk, acc.at[pl.ds(rbase + k * CH, CH)])
    pltpu.sync_copy(srcb.at[pl.ds(s * EPT, EPT)], src_v)
    pltpu.sync_copy(dstb.at[pl.ds(s * EPT, EPT)], dst_v)
    plsc.subcore_barrier()

    def run(h):
        def step(j, carry):
            pltpu.async_copy(h.at[src_v.at[j]], rows, sem).wait()
            pltpu.sync_copy(rows, acc.at[dst_v.at[j]], add=True)
            return carry
        lax.fori_loop(0, EPT, step, 0)

    @pl.when(c == 0)
    def _():
        run(h0)

    @pl.when(c == 1)
    def _():
        run(h1)

    plsc.subcore_barrier()

    @pl.when(c == 0)
    def _():
        pltpu.sync_copy(acc.at[pl.ds(rbase, ROWS_PER_TILE)],
                        out0.at[pl.ds(rbase, ROWS_PER_TILE)])

    @pl.when(c == 1)
    def _():
        pltpu.sync_copy(acc.at[pl.ds(rbase, ROWS_PER_TILE)],
                        out1.at[pl.ds(rbase, ROWS_PER_TILE)])


_spmm = pl.kernel(
    _spmm_body,
    out_type=(
        jax.ShapeDtypeStruct((NP, DH), jnp.float32),
        jax.ShapeDtypeStruct((NP, DH), jnp.float32),
    ),
    mesh=plsc.VectorSubcoreMesh(core_axis_name="c", subcore_axis_name="s"),
    scratch_types=[
        pltpu.VMEM((EPT, CH), jnp.int32),
        pltpu.VMEM((EPT, CH), jnp.int32),
        pltpu.VMEM((CH, DH), jnp.float32),
        pltpu.VMEM_SHARED((NP, DH), jnp.float32),
        pltpu.SemaphoreType.DMA,
    ],
    compiler_params=pltpu.CompilerParams(use_tc_tiling_on_sc=False),
)


_RB = 1024


def _center_body(x_ref, o0_ref, o1_ref):
    x = x_ref[...]
    y = x - jnp.mean(x, axis=1, keepdims=True)
    o0_ref[...] = y[:, :DH]
    o1_ref[...] = y[:, DH:]


_center = pl.pallas_call(
    _center_body,
    grid=(NP // _RB,),
    in_specs=[pl.BlockSpec((_RB, D), lambda i: (i, 0))],
    out_specs=[pl.BlockSpec((_RB, DH), lambda i: (i, 0))] * 2,
    out_shape=(jax.ShapeDtypeStruct((NP, DH), jnp.float32),) * 2,
)


def _final_body(a_ref, b_ref, out_ref, lsm_ref):
    y = jnp.concatenate([a_ref[...], b_ref[...]], axis=1)
    out_ref[...] = y
    m = jnp.max(y, axis=1, keepdims=True)
    ex = jnp.exp(y - m)
    lsm_ref[...] = (y - m) - jnp.log(jnp.sum(ex, axis=1, keepdims=True))


_final = pl.pallas_call(
    _final_body,
    grid=(NP // _RB,),
    in_specs=[pl.BlockSpec((_RB, DH), lambda i: (i, 0))] * 2,
    out_specs=[pl.BlockSpec((_RB, D), lambda i: (i, 0))] * 2,
    out_shape=(jax.ShapeDtypeStruct((NP, D), jnp.float32),) * 2,
)


@jax.jit
def _run(x, edge_index):
    src = edge_index[0]
    dst = edge_index[1]
    pad = E_PAD - E
    srcp = jnp.concatenate([src, jnp.zeros((pad,), jnp.int32)])
    srcp = srcp.reshape(E_PAD // CH, CH)
    dstp = jnp.concatenate([dst, jnp.full((pad,), N, jnp.int32)])
    dstp = dstp.reshape(E_PAD // CH, CH)
    xp = jnp.pad(x, ((0, NP - N), (0, 0)))
    zblk = jnp.zeros((CH, DH), jnp.float32)
    h0, h1 = _center(xp)
    for _ in range(8):
        h0, h1 = _spmm(h0, h1, srcp, dstp, zblk)
    out, lsm = _final(h0, h1)
    return out[:N], lsm[:N]


def kernel(x, edge_index, edge_attr, batch):
    del edge_attr, batch
    return _run(x, edge_index)
